# baseline (speedup 1.0000x reference)
import jax
import jax.numpy as jnp
from jax.experimental import pallas as pl
from jax.experimental.pallas import tpu as pltpu

_BLOCK = 10000
_NBUF = 3


def _linear_kernel(x_hbm, w_ref, b_ref, o_ref, xbuf, sems):
    i = pl.program_id(0)
    nsteps = pl.num_programs(0)
    slot = jax.lax.rem(i, _NBUF)

    def copy_in(step, s):
        return pltpu.make_async_copy(
            x_hbm.at[pl.ds(step * _BLOCK, _BLOCK), :],
            xbuf.at[s],
            sems.at[s],
        )

    @pl.when(i == 0)
    def _():
        for d in range(1, _NBUF):
            copy_in(d, d).start()

    @pl.when(i == 0)
    def _():
        copy_in(0, 0).start()

    copy_in(i, slot).wait()

    acc = jax.lax.dot_general(
        xbuf[slot], w_ref[...],
        dimension_numbers=(((1,), (1,)), ((), ())),
        preferred_element_type=jnp.float32,
    )
    o_ref[...] = acc + b_ref[...]

    nxt = i + _NBUF

    @pl.when(nxt < nsteps)
    def _():
        copy_in(nxt, slot).start()


def kernel(x, W, b):
    n, k = x.shape
    c = W.shape[0]
    b2 = b.reshape(1, c)
    return pl.pallas_call(
        _linear_kernel,
        grid=(n // _BLOCK,),
        in_specs=[
            pl.BlockSpec(memory_space=pl.ANY),
            pl.BlockSpec((c, k), lambda i: (0, 0)),
            pl.BlockSpec((1, c), lambda i: (0, 0)),
        ],
        out_specs=pl.BlockSpec((_BLOCK, c), lambda i: (i, 0)),
        out_shape=jax.ShapeDtypeStruct((n, c), x.dtype),
        scratch_shapes=[
            pltpu.VMEM((_NBUF, _BLOCK, k), jnp.float32),
            pltpu.SemaphoreType.DMA((_NBUF,)),
        ],
        compiler_params=pltpu.CompilerParams(
            dimension_semantics=("arbitrary",),
        ),
    )(x, W, b2)


# speedup vs baseline: 1.0194x; 1.0194x over previous
import jax
import jax.numpy as jnp
from jax.experimental import pallas as pl
from jax.experimental.pallas import tpu as pltpu

_BLOCK = 20000


def _linear_kernel(x_ref, w_ref, b_ref, o_ref):
    acc = jax.lax.dot_general(
        x_ref[...], w_ref[...],
        dimension_numbers=(((1,), (1,)), ((), ())),
        preferred_element_type=jnp.float32,
    )
    o_ref[...] = acc + b_ref[...]


def kernel(x, W, b):
    n, k = x.shape
    c = W.shape[0]
    b2 = b.reshape(1, c)
    return pl.pallas_call(
        _linear_kernel,
        grid=(n // _BLOCK,),
        in_specs=[
            pl.BlockSpec((_BLOCK, k), lambda i: (i, 0)),
            pl.BlockSpec((c, k), lambda i: (0, 0)),
            pl.BlockSpec((1, c), lambda i: (0, 0)),
        ],
        out_specs=pl.BlockSpec((_BLOCK, c), lambda i: (i, 0)),
        out_shape=jax.ShapeDtypeStruct((n, c), x.dtype),
        compiler_params=pltpu.CompilerParams(
            dimension_semantics=("parallel",),
        ),
    )(x, W, b2)
